# baseline (speedup 1.0000x reference)
import functools

import jax
import jax.numpy as jnp
from jax import lax
from jax.experimental import pallas as pl
from jax.experimental.pallas import tpu as pltpu
from jax.experimental.pallas import tpu_sc as plsc

N = 10000
E = 160000
IN = 128
OUT = 32
FE = 4

RPC = 128
NUM_CHUNKS = E // RPC
NW = 32
NTILE = 16
CHUNKS_PER_CORE = NUM_CHUNKS // 2
NP = 10240
TILE_ROWS = NP // NTILE
MW = 128

C = 1600
OIN = OUT * IN


def _mesh():
    return plsc.VectorSubcoreMesh(core_axis_name="c", subcore_axis_name="s")


def _sc_gather_body(nchunks, x_hbm, src_hbm, out_hbm,
                    idx0, idx1, rows0, rows1, sem0, sem1):
    c = lax.axis_index("c")
    s = lax.axis_index("s")
    wid = s * 2 + c

    def fire(k, idx_v, rows_v, sem):
        ch = wid + NW * k

        @pl.when(ch < nchunks)
        def _():
            pltpu.sync_copy(src_hbm.at[pl.ds(ch * RPC, RPC)], idx_v)
            pltpu.async_copy(x_hbm.at[idx_v], rows_v, sem)

    def drain(k, idx_v, rows_v, sem):
        ch = wid + NW * k

        @pl.when(ch < nchunks)
        def _():
            pltpu.make_async_copy(x_hbm.at[idx_v], rows_v, sem).wait()
            pltpu.sync_copy(rows_v, out_hbm.at[pl.ds(ch * RPC, RPC)])

    fire(0, idx0, rows0, sem0)

    def body(t, carry):
        a = 2 * t
        fire(a + 1, idx1, rows1, sem1)
        drain(a, idx0, rows0, sem0)
        fire(a + 2, idx0, rows0, sem0)
        drain(a + 1, idx1, rows1, sem1)
        return carry

    nt = (nchunks + NW - 1) // NW
    lax.fori_loop(0, (nt + 1) // 2, body, 0)
    last = 2 * ((nt + 1) // 2)
    drain(last, idx0, rows0, sem0)


@functools.cache
def _gather_call(es):
    return pl.kernel(
        functools.partial(_sc_gather_body, es // RPC),
        out_type=jax.ShapeDtypeStruct((es, IN), jnp.float32),
        mesh=_mesh(),
        scratch_types=[
            pltpu.VMEM((RPC,), jnp.int32),
            pltpu.VMEM((RPC,), jnp.int32),
            pltpu.VMEM((RPC, IN), jnp.float32),
            pltpu.VMEM((RPC, IN), jnp.float32),
            pltpu.SemaphoreType.DMA,
            pltpu.SemaphoreType.DMA,
        ],
    )


def _sc_scatter_body(nchunks, msg_hbm, dst_hbm, zrows_hbm, agg_hbm,
                     idx_v, msg_v, agg_sh):
    c = lax.axis_index("c")
    s = lax.axis_index("s")
    half = nchunks // 2
    sl = pl.ds(s * TILE_ROWS, TILE_ROWS)
    pltpu.sync_copy(zrows_hbm, agg_sh.at[sl])
    plsc.subcore_barrier()
    base = c * half
    limit = jnp.where(c == 0, half, nchunks - half)

    def body(t, carry):
        ch = s + NTILE * t

        @pl.when(ch < limit)
        def _():
            g = base + ch
            pltpu.sync_copy(dst_hbm.at[pl.ds(g * RPC, RPC)], idx_v.at[0])
            pltpu.sync_copy(msg_hbm.at[pl.ds(g * RPC, RPC)], msg_v)
            pltpu.sync_copy(msg_v, agg_sh.at[idx_v.at[0]], add=True)

        return carry

    lax.fori_loop(0, (nchunks - half + NTILE - 1) // NTILE, body, 0)
    plsc.subcore_barrier()
    pltpu.sync_copy(agg_sh.at[sl], agg_hbm.at[c, sl])


@functools.cache
def _scatter_call(es):
    return pl.kernel(
        functools.partial(_sc_scatter_body, es // RPC),
        out_type=jax.ShapeDtypeStruct((2, NP, MW), jnp.float32),
        mesh=_mesh(),
        scratch_types=[
            pltpu.VMEM((1, RPC), jnp.int32),
            pltpu.VMEM((RPC, MW), jnp.float32),
            pltpu.VMEM_SHARED((NP, MW), jnp.float32),
        ],
    )


NH = 2


def _tc_msg_body(ea_ref, xs_ref, w2_ref, r_ref, out_ref):
    w2 = w2_ref[...]
    r = r_ref[...]
    ohe = (lax.broadcasted_iota(jnp.int32, (1, MW), 1) == OUT
           ).astype(jnp.float32)
    H = C // NH
    for h in range(NH):
        rows = pl.ds(h * H, H)
        ea = ea_ref[rows, :]
        xs = xs_ref[rows, :].astype(jnp.bfloat16)
        z = jnp.dot(ea, w2, preferred_element_type=jnp.float32
                    ).astype(jnp.bfloat16)
        zero = jnp.zeros((), jnp.bfloat16)
        cols = [jnp.maximum(z[:, o * IN:(o + 1) * IN], zero) * xs
                for o in range(OUT)]
        p = jnp.concatenate(cols, axis=1)
        red = jnp.dot(p, r, preferred_element_type=jnp.float32)
        out_ref[rows, :] = red + ohe


def _msg_call(ea5, xsrc, w25, rmat):
    es = ea5.shape[0]
    return pl.pallas_call(
        _tc_msg_body,
        grid=(es // C,),
        in_specs=[
            pl.BlockSpec((C, FE + 1), lambda k: (k, 0)),
            pl.BlockSpec((C, IN), lambda k: (k, 0)),
            pl.BlockSpec((FE + 1, OIN), lambda k: (0, 0)),
            pl.BlockSpec((OIN, MW), lambda k: (0, 0)),
        ],
        out_specs=pl.BlockSpec((C, MW), lambda k: (k, 0)),
        out_shape=jax.ShapeDtypeStruct((es, MW), jnp.float32),
        compiler_params=pltpu.CompilerParams(
            dimension_semantics=("parallel",)),
    )(ea5, xsrc, w25, rmat)


FR = 2000


def _tc_final_body(x_ref, root_ref, bias_ref, *refs):
    agg_refs, out_ref = refs[:-1], refs[-1]
    acc = agg_refs[0][0] + agg_refs[0][1]
    for r in agg_refs[1:]:
        acc = acc + r[0] + r[1]
    agg = acc[:, :OUT]
    cnt = acc[:, OUT:OUT + 1]
    inv = 1.0 / jnp.maximum(cnt, 1.0)
    out_ref[...] = (agg * inv
                    + jnp.dot(x_ref[...], root_ref[...],
                              preferred_element_type=jnp.float32)
                    + bias_ref[...])


def _final_call(x, root, bias2, aggs):
    agg_spec = pl.BlockSpec((2, FR, MW), lambda k: (0, k, 0))
    return pl.pallas_call(
        _tc_final_body,
        grid=(N // FR,),
        in_specs=[
            pl.BlockSpec((FR, IN), lambda k: (k, 0)),
            pl.BlockSpec((IN, OUT), lambda k: (0, 0)),
            pl.BlockSpec((1, OUT), lambda k: (0, 0)),
        ] + [agg_spec] * len(aggs),
        out_specs=pl.BlockSpec((FR, OUT), lambda k: (k, 0)),
        out_shape=jax.ShapeDtypeStruct((N, OUT), jnp.float32),
    )(x, root, bias2, *aggs)


def kernel(x, edge_index, edge_attr, nn_w, nn_b, root, bias):
    src = edge_index[0]
    dst = edge_index[1]
    w2 = nn_w.reshape(IN, OUT, FE).transpose(1, 0, 2).reshape(OIN, FE).T
    b2 = nn_b.reshape(IN, OUT).T.reshape(1, OIN)
    w25 = jnp.concatenate([w2, b2], axis=0).astype(jnp.bfloat16)
    ea5 = jnp.concatenate(
        [edge_attr, jnp.ones((E, 1), jnp.float32)],
        axis=1).astype(jnp.bfloat16)
    rmat = (jnp.arange(OIN)[:, None] // IN
            == jnp.arange(MW)[None, :]).astype(jnp.bfloat16)

    zrows = jnp.zeros((TILE_ROWS, MW), jnp.float32)

    NS = 5
    ES = E // NS
    xsrcs = [_gather_call(ES)(x, src[k * ES:(k + 1) * ES])
             for k in range(NS)]
    msgs = [_msg_call(ea5[k * ES:(k + 1) * ES], xsrcs[k], w25, rmat)
            for k in range(NS)]
    aggs = [_scatter_call(ES)(msgs[k], dst[k * ES:(k + 1) * ES], zrows)
            for k in range(NS)]

    return _final_call(x, root, bias.reshape(1, OUT), aggs)


# speedup vs baseline: 1.0181x; 1.0181x over previous
import functools

import jax
import jax.numpy as jnp
from jax import lax
from jax.experimental import pallas as pl
from jax.experimental.pallas import tpu as pltpu
from jax.experimental.pallas import tpu_sc as plsc

N = 10000
E = 160000
IN = 128
OUT = 32
FE = 4

RPC = 128
NUM_CHUNKS = E // RPC
NW = 32
NTILE = 16
CHUNKS_PER_CORE = NUM_CHUNKS // 2
NP = 10240
TILE_ROWS = NP // NTILE
MW = 128

C = 1600
OIN = OUT * IN


def _mesh():
    return plsc.VectorSubcoreMesh(core_axis_name="c", subcore_axis_name="s")


def _sc_gather_body(nchunks, x_hbm, src_hbm, out_hbm, idx_v, rows_v, sem):
    c = lax.axis_index("c")
    s = lax.axis_index("s")
    wid = s * 2 + c

    def body(t, carry):
        ch = wid + NW * t

        @pl.when(ch < nchunks)
        def _():
            pltpu.sync_copy(src_hbm.at[pl.ds(ch * RPC, RPC)], idx_v)
            pltpu.async_copy(x_hbm.at[idx_v], rows_v, sem).wait()
            pltpu.sync_copy(rows_v, out_hbm.at[pl.ds(ch * RPC, RPC)])

        return carry

    lax.fori_loop(0, (nchunks + NW - 1) // NW, body, 0)


@functools.cache
def _gather_call(es):
    return pl.kernel(
        functools.partial(_sc_gather_body, es // RPC),
        out_type=jax.ShapeDtypeStruct((es, IN), jnp.float32),
        mesh=_mesh(),
        scratch_types=[
            pltpu.VMEM((RPC,), jnp.int32),
            pltpu.VMEM((RPC, IN), jnp.float32),
            pltpu.SemaphoreType.DMA,
        ],
    )


def _sc_scatter_body(nchunks, msg_hbm, dst_hbm, zrows_hbm, agg_hbm,
                     idx_v, msg_v, agg_sh):
    c = lax.axis_index("c")
    s = lax.axis_index("s")
    half = nchunks // 2
    sl = pl.ds(s * TILE_ROWS, TILE_ROWS)
    pltpu.sync_copy(zrows_hbm, agg_sh.at[sl])
    plsc.subcore_barrier()
    base = c * half
    limit = jnp.where(c == 0, half, nchunks - half)

    def body(t, carry):
        ch = s + NTILE * t

        @pl.when(ch < limit)
        def _():
            g = base + ch
            pltpu.sync_copy(dst_hbm.at[pl.ds(g * RPC, RPC)], idx_v.at[0])
            pltpu.sync_copy(msg_hbm.at[pl.ds(g * RPC, RPC)], msg_v)
            pltpu.sync_copy(msg_v, agg_sh.at[idx_v.at[0]], add=True)

        return carry

    lax.fori_loop(0, (nchunks - half + NTILE - 1) // NTILE, body, 0)
    plsc.subcore_barrier()
    pltpu.sync_copy(agg_sh.at[sl], agg_hbm.at[c, sl])


@functools.cache
def _scatter_call(es):
    return pl.kernel(
        functools.partial(_sc_scatter_body, es // RPC),
        out_type=jax.ShapeDtypeStruct((2, NP, MW), jnp.float32),
        mesh=_mesh(),
        scratch_types=[
            pltpu.VMEM((1, RPC), jnp.int32),
            pltpu.VMEM((RPC, MW), jnp.float32),
            pltpu.VMEM_SHARED((NP, MW), jnp.float32),
        ],
    )


NH = 2


def _tc_msg_body(ea_ref, xs_ref, w2_ref, r_ref, out_ref):
    w2 = w2_ref[...]
    r = r_ref[...]
    ohe = (lax.broadcasted_iota(jnp.int32, (1, MW), 1) == OUT
           ).astype(jnp.float32)
    H = C // NH
    for h in range(NH):
        rows = pl.ds(h * H, H)
        ea = ea_ref[rows, :]
        xs = xs_ref[rows, :].astype(jnp.bfloat16)
        z = jnp.dot(ea, w2, preferred_element_type=jnp.float32
                    ).astype(jnp.bfloat16)
        zero = jnp.zeros((), jnp.bfloat16)
        cols = [jnp.maximum(z[:, o * IN:(o + 1) * IN], zero) * xs
                for o in range(OUT)]
        p = jnp.concatenate(cols, axis=1)
        red = jnp.dot(p, r, preferred_element_type=jnp.float32)
        out_ref[rows, :] = red + ohe


def _msg_call(ea5, xsrc, w25, rmat):
    es = ea5.shape[0]
    return pl.pallas_call(
        _tc_msg_body,
        grid=(es // C,),
        in_specs=[
            pl.BlockSpec((C, FE + 1), lambda k: (k, 0)),
            pl.BlockSpec((C, IN), lambda k: (k, 0)),
            pl.BlockSpec((FE + 1, OIN), lambda k: (0, 0)),
            pl.BlockSpec((OIN, MW), lambda k: (0, 0)),
        ],
        out_specs=pl.BlockSpec((C, MW), lambda k: (k, 0)),
        out_shape=jax.ShapeDtypeStruct((es, MW), jnp.float32),
        compiler_params=pltpu.CompilerParams(
            dimension_semantics=("parallel",)),
    )(ea5, xsrc, w25, rmat)


FR = 2000


def _tc_final_body(x_ref, root_ref, bias_ref, *refs):
    agg_refs, out_ref = refs[:-1], refs[-1]
    acc = agg_refs[0][0] + agg_refs[0][1]
    for r in agg_refs[1:]:
        acc = acc + r[0] + r[1]
    agg = acc[:, :OUT]
    cnt = acc[:, OUT:OUT + 1]
    inv = 1.0 / jnp.maximum(cnt, 1.0)
    out_ref[...] = (agg * inv
                    + jnp.dot(x_ref[...], root_ref[...],
                              preferred_element_type=jnp.float32)
                    + bias_ref[...])


def _final_call(x, root, bias2, aggs):
    agg_spec = pl.BlockSpec((2, FR, MW), lambda k: (0, k, 0))
    return pl.pallas_call(
        _tc_final_body,
        grid=(N // FR,),
        in_specs=[
            pl.BlockSpec((FR, IN), lambda k: (k, 0)),
            pl.BlockSpec((IN, OUT), lambda k: (0, 0)),
            pl.BlockSpec((1, OUT), lambda k: (0, 0)),
        ] + [agg_spec] * len(aggs),
        out_specs=pl.BlockSpec((FR, OUT), lambda k: (k, 0)),
        out_shape=jax.ShapeDtypeStruct((N, OUT), jnp.float32),
    )(x, root, bias2, *aggs)


def kernel(x, edge_index, edge_attr, nn_w, nn_b, root, bias):
    src = edge_index[0]
    dst = edge_index[1]
    w2 = nn_w.reshape(IN, OUT, FE).transpose(1, 0, 2).reshape(OIN, FE).T
    b2 = nn_b.reshape(IN, OUT).T.reshape(1, OIN)
    w25 = jnp.concatenate([w2, b2], axis=0).astype(jnp.bfloat16)
    ea5 = jnp.concatenate(
        [edge_attr, jnp.ones((E, 1), jnp.float32)],
        axis=1).astype(jnp.bfloat16)
    rmat = (jnp.arange(OIN)[:, None] // IN
            == jnp.arange(MW)[None, :]).astype(jnp.bfloat16)

    zrows = jnp.zeros((TILE_ROWS, MW), jnp.float32)

    NS = 5
    ES = E // NS
    xsrcs = [_gather_call(ES)(x, src[k * ES:(k + 1) * ES])
             for k in range(NS)]
    msgs = [_msg_call(ea5[k * ES:(k + 1) * ES], xsrcs[k], w25, rmat)
            for k in range(NS)]
    aggs = [_scatter_call(ES)(msgs[k], dst[k * ES:(k + 1) * ES], zrows)
            for k in range(NS)]

    return _final_call(x, root, bias.reshape(1, OUT), aggs)
